# baseline (speedup 1.0000x reference)
import functools

import jax
import jax.numpy as jnp
from jax.experimental import pallas as pl

B, N, L = 1, 8, 512
D_MSA, D_PAIR, D_STATE = 256, 128, 16
L0_IN, L0_OUT, D_EDGE = 32, 16, 32
D_HID = 128
H_MSG = 64
TOPK = 64

BI = 64
BJ = 128
NI = L // BI
NJ = L // BJ


def _ln(x, g, b, eps=1e-5):
    m = jnp.mean(x, axis=-1, keepdims=True)
    v = jnp.var(x, axis=-1, keepdims=True)
    return (x - m) / jnp.sqrt(v + eps) * g + b


def _prep_kernel(ca_ref, caT_ref, msa0_ref, state_ref,
                 g_msa_ref, b_msa_ref, g_state_ref, b_state_ref,
                 wxa_ref, wxb_ref, bbx_ref, g_node_ref, b_node_ref,
                 dist_ref, maskf_ref, node_ref):
    ca = ca_ref[...]
    caT = caT_ref[...]
    dx = ca[:, 0:1] - caT[0:1, :]
    dy = ca[:, 1:2] - caT[1:2, :]
    dz = ca[:, 2:3] - caT[2:3, :]
    dist = jnp.sqrt(dx * dx + dy * dy + dz * dz + 1e-8)
    dist_ref[...] = dist

    bits = jax.lax.bitcast_convert_type(dist, jnp.int32)
    lo0 = jnp.min(bits, axis=1, keepdims=True)
    hi0 = jnp.max(bits, axis=1, keepdims=True)

    def body(_, carry):
        lo, hi = carry
        mid = lo + (hi - lo) // 2
        cnt = jnp.sum((bits <= mid).astype(jnp.float32), axis=1, keepdims=True)
        ge = cnt >= float(TOPK)
        return jnp.where(ge, lo, mid + 1), jnp.where(ge, mid, hi)

    lo, hi = jax.lax.fori_loop(0, 31, body, (lo0, hi0))
    v = lo
    mask_lt = bits < v
    cnt_lt = jnp.sum(mask_lt.astype(jnp.float32), axis=1, keepdims=True)
    tie = bits == v
    tie_f = tie.astype(jnp.float32)
    ii = jax.lax.broadcasted_iota(jnp.int32, (L, L), 0)
    jj = jax.lax.broadcasted_iota(jnp.int32, (L, L), 1)
    M = (ii <= jj).astype(jnp.float32)
    cs = jnp.dot(tie_f, M, preferred_element_type=jnp.float32)
    mask_tie = tie & (cs <= (float(TOPK) - cnt_lt))
    maskf_ref[...] = jnp.where(mask_lt | mask_tie, 1.0, 0.0)

    mln = _ln(msa0_ref[...], g_msa_ref[...], b_msa_ref[...])
    sln = _ln(state_ref[...], g_state_ref[...], b_state_ref[...])
    x = (jnp.dot(mln, wxa_ref[...], preferred_element_type=jnp.float32)
         + jnp.dot(sln, wxb_ref[...], preferred_element_type=jnp.float32)
         + bbx_ref[...])
    node_ref[...] = _ln(x, g_node_ref[...], b_node_ref[...])


def _pair_kernel(pair_ref, dist_ref, maskf_ref, nodei_ref, nodej_ref,
                 idxc_ref, idxr_ref,
                 wcat1_ref, onesp_ref, v1_ref, c1_ref,
                 wcat2_ref, onese_ref, v2_ref, c2_ref,
                 we2b_ref, we2c_ref,
                 wcat3_ref, v3_ref, c3_ref,
                 wmi_ref, wmj_ref,
                 acc_ref):
    j = pl.program_id(1)

    x = pair_ref[0].reshape(BI * BJ, D_PAIR)
    a = jnp.dot(x, wcat1_ref[...], preferred_element_type=jnp.float32)
    m = a[:, D_EDGE:D_EDGE + 1]
    msq = jnp.dot(x * x, onesp_ref[...], preferred_element_type=jnp.float32)
    r = jax.lax.rsqrt(msq - m * m + 1e-5)
    e1 = r * a[:, :D_EDGE] - (m * r) * v1_ref[...] + c1_ref[...]

    a2 = jnp.dot(e1, wcat2_ref[...], preferred_element_type=jnp.float32)
    m1 = a2[:, D_EDGE:D_EDGE + 1]
    msq1 = jnp.dot(e1 * e1, onese_ref[...], preferred_element_type=jnp.float32)
    r1 = jax.lax.rsqrt(msq1 - m1 * m1 + 1e-5)
    e2a = r1 * a2[:, :D_EDGE] - (m1 * r1) * v2_ref[...] + c2_ref[...]

    d = dist_ref[...]
    kidx = jax.lax.broadcasted_iota(jnp.int32, (1, 1, 36), 2).astype(jnp.float32)
    centers = 2.0 + kidx * (20.0 / 35.0)
    sigma = 20.0 / 36.0
    rbf = jnp.exp(-(((d[..., None] - centers) / sigma) ** 2))

    off = idxr_ref[...] - idxc_ref[...]
    seqsep = jnp.sign(off) * jnp.log(jnp.abs(off) + 1.0)

    e2f = (e2a + jnp.dot(rbf.reshape(BI * BJ, 36), we2b_ref[...],
                         preferred_element_type=jnp.float32))
    e2 = (e2f.reshape(BI, BJ, D_EDGE)
          + seqsep[..., None] * we2c_ref[...]).reshape(BI * BJ, D_EDGE)

    a3 = jnp.dot(e2, wcat3_ref[...], preferred_element_type=jnp.float32)
    m2 = a3[:, H_MSG:H_MSG + 1]
    msq2 = jnp.dot(e2 * e2, onese_ref[...], preferred_element_type=jnp.float32)
    r2 = jax.lax.rsqrt(msq2 - m2 * m2 + 1e-5)
    me = r2 * a3[:, :H_MSG] - (m2 * r2) * v3_ref[...] + c3_ref[...]

    mi = jnp.dot(nodei_ref[...], wmi_ref[...],
                 preferred_element_type=jnp.float32)
    mj = jnp.dot(nodej_ref[...], wmj_ref[...],
                 preferred_element_type=jnp.float32)
    msg = jax.nn.relu(me.reshape(BI, BJ, H_MSG)
                      + mi[:, None, :] + mj[None, :, :])
    msg = msg * maskf_ref[...][..., None]
    partial = jnp.sum(msg, axis=1)

    @pl.when(j == 0)
    def _():
        acc_ref[...] = jnp.zeros_like(acc_ref)

    acc_ref[...] += partial


def _head_kernel(agg_ref, msa0_ref, r9_ref, tin_ref,
                 wst_ref, bbst_ref, woff_ref, bboff_ref,
                 g_s0_ref, b_s0_ref, g_si_ref, b_si_ref,
                 ws0_ref, bbs0_ref, wsi_ref, bbsi_ref,
                 wl1_ref, bl1_ref, wl2_ref, bl2_ref,
                 wl3_ref, bl3_ref, wl4_ref, bl4_ref,
                 wout_ref, bout_ref,
                 t_ref, ns_ref, alpha_ref):
    agg = agg_ref[...] * (1.0 / TOPK)
    ns = jnp.dot(agg, wst_ref[...],
                 preferred_element_type=jnp.float32) + bbst_ref[...]
    ns_ref[...] = ns
    off6 = jnp.dot(agg, woff_ref[...],
                   preferred_element_type=jnp.float32) + bboff_ref[...]
    delT = off6[:, 0:3] / 10.0
    r9 = r9_ref[...]
    t0 = jnp.sum(r9[:, 0:3] * delT, axis=1, keepdims=True)
    t1 = jnp.sum(r9[:, 3:6] * delT, axis=1, keepdims=True)
    t2 = jnp.sum(r9[:, 6:9] * delT, axis=1, keepdims=True)
    t_ref[...] = jnp.concatenate([t0, t1, t2], axis=1) + tin_ref[...]

    s0 = _ln(msa0_ref[...], g_s0_ref[...], b_s0_ref[...])
    si_in = _ln(ns, g_si_ref[...], b_si_ref[...])
    si = (jnp.dot(s0, ws0_ref[...], preferred_element_type=jnp.float32)
          + bbs0_ref[...]
          + jnp.dot(si_in, wsi_ref[...], preferred_element_type=jnp.float32)
          + bbsi_ref[...])
    h = jax.nn.relu(jnp.dot(jax.nn.relu(si), wl1_ref[...],
                            preferred_element_type=jnp.float32) + bl1_ref[...])
    si = si + jnp.dot(h, wl2_ref[...],
                      preferred_element_type=jnp.float32) + bl2_ref[...]
    h = jax.nn.relu(jnp.dot(jax.nn.relu(si), wl3_ref[...],
                            preferred_element_type=jnp.float32) + bl3_ref[...])
    si = si + jnp.dot(h, wl4_ref[...],
                      preferred_element_type=jnp.float32) + bl4_ref[...]
    alpha_ref[...] = (jnp.dot(jax.nn.relu(si), wout_ref[...],
                              preferred_element_type=jnp.float32)
                      + bout_ref[...])


def kernel(msa, pair, R_in, T_in, xyz, state, idx, motif_mask, top_k, g_msa_ln, b_msa_ln, g_pair_ln, b_pair_ln, g_state_ln, b_state_ln, g_node_ln, b_node_ln, g_e1_ln, b_e1_ln, g_e2_ln, b_e2_ln, g_s0_ln, b_s0_ln, g_si_ln, b_si_ln, W_x, bb_x, W_e1, bb_e1, W_e2, bb_e2, W_msg, bb_msg, W_st, bb_st, W_off, bb_off, W_s0, bb_s0, W_si, bb_si, W_l1, bb_l1, W_l2, bb_l2, W_l3, bb_l3, W_l4, bb_l4, W_out, bb_out):
    f32 = jnp.float32
    msa0 = msa[0, 0]
    ca = xyz[0, :, 1, :]
    caT = jnp.transpose(ca)
    state0 = state[0]
    idxf = idx[0].astype(f32)
    idx_col = idxf[:, None]
    idx_row = idxf[None, :]
    r9 = R_in[0].reshape(L, 9)
    tin = T_in[0]

    row = lambda a: a.reshape(1, -1)

    dist, maskf, node = pl.pallas_call(
        _prep_kernel,
        out_shape=[
            jax.ShapeDtypeStruct((L, L), f32),
            jax.ShapeDtypeStruct((L, L), f32),
            jax.ShapeDtypeStruct((L, L0_IN), f32),
        ],
    )(ca, caT, msa0, state0,
      row(g_msa_ln), row(b_msa_ln), row(g_state_ln), row(b_state_ln),
      W_x[:D_MSA], W_x[D_MSA:], row(bb_x), row(g_node_ln), row(b_node_ln))

    wspec = lambda shp: pl.BlockSpec(shp, lambda i, j: (0,) * len(shp))
    pair_in_specs = [
        pl.BlockSpec((1, BI, BJ, D_PAIR), lambda i, j: (0, i, j, 0)),
        pl.BlockSpec((BI, BJ), lambda i, j: (i, j)),
        pl.BlockSpec((BI, BJ), lambda i, j: (i, j)),
        pl.BlockSpec((BI, L0_IN), lambda i, j: (i, 0)),
        pl.BlockSpec((BJ, L0_IN), lambda i, j: (j, 0)),
        pl.BlockSpec((BI, 1), lambda i, j: (i, 0)),
        pl.BlockSpec((1, BJ), lambda i, j: (0, j)),
        wspec((D_PAIR, D_EDGE + 1)), wspec((D_PAIR, 1)),
        wspec((1, D_EDGE)), wspec((1, D_EDGE)),
        wspec((D_EDGE, D_EDGE + 1)), wspec((D_EDGE, 1)),
        wspec((1, D_EDGE)), wspec((1, D_EDGE)),
        wspec((36, D_EDGE)), wspec((1, D_EDGE)),
        wspec((D_EDGE, H_MSG + 1)), wspec((1, H_MSG)), wspec((1, H_MSG)),
        wspec((L0_IN, H_MSG)), wspec((L0_IN, H_MSG)),
    ]
    We2a = W_e2[:D_EDGE]
    We2b = W_e2[D_EDGE:D_EDGE + 36]
    We2c = W_e2[D_EDGE + 36:]
    Wme = W_msg[2 * L0_IN:]
    wcat1 = jnp.concatenate(
        [g_pair_ln[:, None] * W_e1,
         jnp.full((D_PAIR, 1), 1.0 / D_PAIR, f32)], axis=1)
    onesp = jnp.full((D_PAIR, 1), 1.0 / D_PAIR, f32)
    v1 = (g_pair_ln @ W_e1)[None, :]
    c1 = (b_pair_ln @ W_e1 + bb_e1)[None, :]
    wcat2 = jnp.concatenate(
        [g_e1_ln[:, None] * We2a,
         jnp.full((D_EDGE, 1), 1.0 / D_EDGE, f32)], axis=1)
    onese = jnp.full((D_EDGE, 1), 1.0 / D_EDGE, f32)
    v2 = (g_e1_ln @ We2a)[None, :]
    c2 = (b_e1_ln @ We2a + bb_e2)[None, :]
    wcat3 = jnp.concatenate(
        [g_e2_ln[:, None] * Wme,
         jnp.full((D_EDGE, 1), 1.0 / D_EDGE, f32)], axis=1)
    v3 = (g_e2_ln @ Wme)[None, :]
    c3 = (b_e2_ln @ Wme + bb_msg)[None, :]
    agg = pl.pallas_call(
        _pair_kernel,
        grid=(NI, NJ),
        in_specs=pair_in_specs,
        out_specs=pl.BlockSpec((BI, H_MSG), lambda i, j: (i, 0)),
        out_shape=jax.ShapeDtypeStruct((L, H_MSG), f32),
    )(pair, dist, maskf, node, node, idx_col, idx_row,
      wcat1, onesp, v1, c1,
      wcat2, onese, v2, c2,
      We2b, We2c,
      wcat3, v3, c3,
      W_msg[:L0_IN], W_msg[L0_IN:2 * L0_IN])

    T, new_state, alpha = pl.pallas_call(
        _head_kernel,
        out_shape=[
            jax.ShapeDtypeStruct((L, 3), f32),
            jax.ShapeDtypeStruct((L, L0_OUT), f32),
            jax.ShapeDtypeStruct((L, 20), f32),
        ],
    )(agg, msa0, r9, tin,
      W_st, row(bb_st), W_off, row(bb_off),
      row(g_s0_ln), row(b_s0_ln), row(g_si_ln), row(b_si_ln),
      W_s0, row(bb_s0), W_si, row(bb_si),
      W_l1, row(bb_l1), W_l2, row(bb_l2),
      W_l3, row(bb_l3), W_l4, row(bb_l4),
      W_out, row(bb_out))

    return (R_in, T[None], new_state[None], alpha.reshape(1, L, 10, 2))


# speedup vs baseline: 1.9521x; 1.9521x over previous
import functools

import jax
import jax.numpy as jnp
from jax import lax
from jax.experimental import pallas as pl
from jax.experimental.pallas import tpu as pltpu
from jax.experimental.pallas import tpu_sc as plsc

B, N, L = 1, 8, 512
D_MSA, D_PAIR, D_STATE = 256, 128, 16
L0_IN, L0_OUT, D_EDGE = 32, 16, 32
D_HID = 128
H_MSG = 64
TOPK = 64

NW = 32
ROWS_PER_W = L // NW
BI = 64
NBLK = L // BI
BJC = 128
NJC = L // BJC


def _ln(x, g, b, eps=1e-5):
    m = jnp.mean(x, axis=-1, keepdims=True)
    v = jnp.var(x, axis=-1, keepdims=True)
    return (x - m) / jnp.sqrt(v + eps) * g + b


def _prep_kernel(ca_ref, caT_ref, msa0_ref, state_ref,
                 g_msa_ref, b_msa_ref, g_state_ref, b_state_ref,
                 wxa_ref, wxb_ref, bbx_ref, g_node_ref, b_node_ref,
                 dist_ref, maskf_ref, csum_ref, node_ref):
    ca = ca_ref[...]
    caT = caT_ref[...]
    dx = ca[:, 0:1] - caT[0:1, :]
    dy = ca[:, 1:2] - caT[1:2, :]
    dz = ca[:, 2:3] - caT[2:3, :]
    dist = jnp.sqrt(dx * dx + dy * dy + dz * dz + 1e-8)
    dist_ref[...] = dist

    bits = jax.lax.bitcast_convert_type(dist, jnp.int32)
    lo0 = jnp.min(bits, axis=1, keepdims=True)
    hi0 = jnp.max(bits, axis=1, keepdims=True)

    def body(_, carry):
        lo, hi = carry
        mid = lo + (hi - lo) // 2
        cnt = jnp.sum((bits <= mid).astype(jnp.float32), axis=1, keepdims=True)
        ge = cnt >= float(TOPK)
        return jnp.where(ge, lo, mid + 1), jnp.where(ge, mid, hi)

    lo, hi = jax.lax.fori_loop(0, 31, body, (lo0, hi0))
    v = lo
    mask_lt = bits < v
    cnt_lt = jnp.sum(mask_lt.astype(jnp.float32), axis=1, keepdims=True)
    tie = bits == v
    tie_f = tie.astype(jnp.float32)
    ii = jax.lax.broadcasted_iota(jnp.int32, (L, L), 0)
    jj = jax.lax.broadcasted_iota(jnp.int32, (L, L), 1)
    M = (ii <= jj).astype(jnp.float32)
    cs_tie = jnp.dot(tie_f, M, preferred_element_type=jnp.float32)
    mask_tie = tie & (cs_tie <= (float(TOPK) - cnt_lt))
    maskf = jnp.where(mask_lt | mask_tie, 1.0, 0.0)
    maskf_ref[...] = maskf
    csum_ref[...] = jnp.dot(maskf, M, preferred_element_type=jnp.float32)

    mln = _ln(msa0_ref[...], g_msa_ref[...], b_msa_ref[...])
    sln = _ln(state_ref[...], g_state_ref[...], b_state_ref[...])
    x = (jnp.dot(mln, wxa_ref[...], preferred_element_type=jnp.float32)
         + jnp.dot(sln, wxb_ref[...], preferred_element_type=jnp.float32)
         + bbx_ref[...])
    nodev = _ln(x, g_node_ref[...], b_node_ref[...])
    node_ref[...] = jnp.concatenate(
        [nodev, jnp.zeros((L, D_PAIR - L0_IN), jnp.float32)], axis=1)


def _compact_kernel(c_ref, m_ref, d_ref,
                    nbrf_ref, distk_ref, nbri_ref, flat_ref):
    jb = pl.program_id(0)
    c = c_ref[...]
    mf = m_ref[...]
    d = d_ref[...]
    kvec1 = 1.0 + jax.lax.broadcasted_iota(
        jnp.int32, (1, 1, TOPK), 2).astype(jnp.float32)
    jvals = (jb * BJC + jax.lax.broadcasted_iota(
        jnp.int32, (1, BJC, 1), 1)).astype(jnp.float32)
    oh = jnp.where(c[:, :, None] == kvec1, mf[:, :, None], 0.0)

    @pl.when(jb == 0)
    def _():
        nbrf_ref[...] = jnp.zeros_like(nbrf_ref)
        distk_ref[...] = jnp.zeros_like(distk_ref)

    nbrf_ref[...] += jnp.sum(jvals * oh, axis=1)
    distk_ref[...] += jnp.sum(d[:, :, None] * oh, axis=1)

    @pl.when(jb == NJC - 1)
    def _():
        nbri = nbrf_ref[...].astype(jnp.int32)
        nbri_ref[...] = nbri
        rowbase = jax.lax.broadcasted_iota(jnp.int32, (L, TOPK), 0) * L
        flat_ref[...] = nbri + rowbase


def _sc_gather_kernel(flat_hbm, nbri_hbm, pairflat_hbm, node_hbm,
                      pairk_hbm, nodek_hbm,
                      idxbuf, nbrbuf, prows, nrows, sem):
    c = lax.axis_index("c")
    s = lax.axis_index("s")
    wid = s * 2 + c

    def row_body(t, carry):
        i = wid * ROWS_PER_W + t
        pltpu.sync_copy(flat_hbm.at[i], idxbuf)
        pltpu.sync_copy(nbri_hbm.at[i], nbrbuf)
        cp1 = pltpu.async_copy(pairflat_hbm.at[idxbuf], prows, sem)
        cp1.wait()
        cp2 = pltpu.async_copy(node_hbm.at[nbrbuf], nrows, sem)
        cp2.wait()
        pltpu.sync_copy(prows, pairk_hbm.at[pl.ds(i * TOPK, TOPK)])
        pltpu.sync_copy(nrows, nodek_hbm.at[pl.ds(i * TOPK, TOPK)])
        return carry

    lax.fori_loop(0, ROWS_PER_W, row_body, 0)


def _edge_kernel(pairk_ref, nodek_ref, distk_ref, nbri_ref,
                 nodei_ref, idxc_ref,
                 g_pair_ref, b_pair_ref, we1_ref, bbe1_ref, g_e1_ref, b_e1_ref,
                 we2a_ref, we2b_ref, we2c_ref, bbe2_ref, g_e2_ref, b_e2_ref,
                 wmi_ref, wmj_ref, wme_ref, bbm_ref,
                 agg_ref):
    x = pairk_ref[...]
    pn = _ln(x, g_pair_ref[...], b_pair_ref[...])
    e1 = (jnp.dot(pn, we1_ref[...], preferred_element_type=jnp.float32)
          + bbe1_ref[...])
    e1 = _ln(e1, g_e1_ref[...], b_e1_ref[...])

    d = distk_ref[...]
    kidx = jax.lax.broadcasted_iota(jnp.int32, (1, 1, 36), 2).astype(jnp.float32)
    centers = 2.0 + kidx * (20.0 / 35.0)
    sigma = 20.0 / 36.0
    rbf = jnp.exp(-(((d[..., None] - centers) / sigma) ** 2))

    off = nbri_ref[...].astype(jnp.float32) - idxc_ref[...]
    seqsep = jnp.sign(off) * jnp.log(jnp.abs(off) + 1.0)

    e2f = (jnp.dot(e1, we2a_ref[...], preferred_element_type=jnp.float32)
           + jnp.dot(rbf.reshape(BI * TOPK, 36), we2b_ref[...],
                     preferred_element_type=jnp.float32)
           + bbe2_ref[...])
    e2 = e2f.reshape(BI, TOPK, D_EDGE) + seqsep[..., None] * we2c_ref[...]
    e = _ln(e2, g_e2_ref[...], b_e2_ref[...]).reshape(BI * TOPK, D_EDGE)

    mi = jnp.dot(nodei_ref[:, :L0_IN], wmi_ref[...],
                 preferred_element_type=jnp.float32)
    mj = jnp.dot(nodek_ref[:, :L0_IN], wmj_ref[...],
                 preferred_element_type=jnp.float32)
    me = jnp.dot(e, wme_ref[...], preferred_element_type=jnp.float32)
    msg = jax.nn.relu(me.reshape(BI, TOPK, H_MSG)
                      + mi[:, None, :]
                      + mj.reshape(BI, TOPK, H_MSG)
                      + bbm_ref[...])
    agg_ref[...] = jnp.sum(msg, axis=1)


def _head_kernel(agg_ref, msa0_ref, r9_ref, tin_ref,
                 wst_ref, bbst_ref, woff_ref, bboff_ref,
                 g_s0_ref, b_s0_ref, g_si_ref, b_si_ref,
                 ws0_ref, bbs0_ref, wsi_ref, bbsi_ref,
                 wl1_ref, bl1_ref, wl2_ref, bl2_ref,
                 wl3_ref, bl3_ref, wl4_ref, bl4_ref,
                 wout_ref, bout_ref,
                 t_ref, ns_ref, alpha_ref):
    agg = agg_ref[...] * (1.0 / TOPK)
    ns = jnp.dot(agg, wst_ref[...],
                 preferred_element_type=jnp.float32) + bbst_ref[...]
    ns_ref[...] = ns
    off6 = jnp.dot(agg, woff_ref[...],
                   preferred_element_type=jnp.float32) + bboff_ref[...]
    delT = off6[:, 0:3] / 10.0
    r9 = r9_ref[...]
    t0 = jnp.sum(r9[:, 0:3] * delT, axis=1, keepdims=True)
    t1 = jnp.sum(r9[:, 3:6] * delT, axis=1, keepdims=True)
    t2 = jnp.sum(r9[:, 6:9] * delT, axis=1, keepdims=True)
    t_ref[...] = jnp.concatenate([t0, t1, t2], axis=1) + tin_ref[...]

    s0 = _ln(msa0_ref[...], g_s0_ref[...], b_s0_ref[...])
    si_in = _ln(ns, g_si_ref[...], b_si_ref[...])
    si = (jnp.dot(s0, ws0_ref[...], preferred_element_type=jnp.float32)
          + bbs0_ref[...]
          + jnp.dot(si_in, wsi_ref[...], preferred_element_type=jnp.float32)
          + bbsi_ref[...])
    h = jax.nn.relu(jnp.dot(jax.nn.relu(si), wl1_ref[...],
                            preferred_element_type=jnp.float32) + bl1_ref[...])
    si = si + jnp.dot(h, wl2_ref[...],
                      preferred_element_type=jnp.float32) + bl2_ref[...]
    h = jax.nn.relu(jnp.dot(jax.nn.relu(si), wl3_ref[...],
                            preferred_element_type=jnp.float32) + bl3_ref[...])
    si = si + jnp.dot(h, wl4_ref[...],
                      preferred_element_type=jnp.float32) + bl4_ref[...]
    alpha_ref[...] = (jnp.dot(jax.nn.relu(si), wout_ref[...],
                              preferred_element_type=jnp.float32)
                      + bout_ref[...])


def kernel(msa, pair, R_in, T_in, xyz, state, idx, motif_mask, top_k, g_msa_ln, b_msa_ln, g_pair_ln, b_pair_ln, g_state_ln, b_state_ln, g_node_ln, b_node_ln, g_e1_ln, b_e1_ln, g_e2_ln, b_e2_ln, g_s0_ln, b_s0_ln, g_si_ln, b_si_ln, W_x, bb_x, W_e1, bb_e1, W_e2, bb_e2, W_msg, bb_msg, W_st, bb_st, W_off, bb_off, W_s0, bb_s0, W_si, bb_si, W_l1, bb_l1, W_l2, bb_l2, W_l3, bb_l3, W_l4, bb_l4, W_out, bb_out):
    f32 = jnp.float32
    msa0 = msa[0, 0]
    ca = xyz[0, :, 1, :]
    caT = jnp.transpose(ca)
    state0 = state[0]
    idx_col = idx[0].astype(f32)[:, None]
    r9 = R_in[0].reshape(L, 9)
    tin = T_in[0]
    pairflat = pair.reshape(L * L, D_PAIR)

    row = lambda a: a.reshape(1, -1)

    dist, maskf, csum, node = pl.pallas_call(
        _prep_kernel,
        out_shape=[
            jax.ShapeDtypeStruct((L, L), f32),
            jax.ShapeDtypeStruct((L, L), f32),
            jax.ShapeDtypeStruct((L, L), f32),
            jax.ShapeDtypeStruct((L, D_PAIR), f32),
        ],
    )(ca, caT, msa0, state0,
      row(g_msa_ln), row(b_msa_ln), row(g_state_ln), row(b_state_ln),
      W_x[:D_MSA], W_x[D_MSA:], row(bb_x), row(g_node_ln), row(b_node_ln))

    nbrf, distk, nbri, flatidx = pl.pallas_call(
        _compact_kernel,
        grid=(NJC,),
        in_specs=[
            pl.BlockSpec((L, BJC), lambda j: (0, j)),
            pl.BlockSpec((L, BJC), lambda j: (0, j)),
            pl.BlockSpec((L, BJC), lambda j: (0, j)),
        ],
        out_specs=[
            pl.BlockSpec((L, TOPK), lambda j: (0, 0)),
            pl.BlockSpec((L, TOPK), lambda j: (0, 0)),
            pl.BlockSpec((L, TOPK), lambda j: (0, 0)),
            pl.BlockSpec((L, TOPK), lambda j: (0, 0)),
        ],
        out_shape=[
            jax.ShapeDtypeStruct((L, TOPK), f32),
            jax.ShapeDtypeStruct((L, TOPK), f32),
            jax.ShapeDtypeStruct((L, TOPK), jnp.int32),
            jax.ShapeDtypeStruct((L, TOPK), jnp.int32),
        ],
    )(csum, maskf, dist)

    sc_gather = functools.partial(
        pl.kernel,
        mesh=plsc.VectorSubcoreMesh(core_axis_name="c", subcore_axis_name="s"),
        out_type=[
            jax.ShapeDtypeStruct((L * TOPK, D_PAIR), f32),
            jax.ShapeDtypeStruct((L * TOPK, D_PAIR), f32),
        ],
        scratch_types=[
            pltpu.VMEM((TOPK,), jnp.int32),
            pltpu.VMEM((TOPK,), jnp.int32),
            pltpu.VMEM((TOPK, D_PAIR), f32),
            pltpu.VMEM((TOPK, D_PAIR), f32),
            pltpu.SemaphoreType.DMA,
        ],
    )(_sc_gather_kernel)
    pairk, nodek = sc_gather(flatidx, nbri, pairflat, node)

    wspec = lambda shp: pl.BlockSpec(shp, lambda i: (0,) * len(shp))
    agg = pl.pallas_call(
        _edge_kernel,
        grid=(NBLK,),
        in_specs=[
            pl.BlockSpec((BI * TOPK, D_PAIR), lambda i: (i, 0)),
            pl.BlockSpec((BI * TOPK, D_PAIR), lambda i: (i, 0)),
            pl.BlockSpec((BI, TOPK), lambda i: (i, 0)),
            pl.BlockSpec((BI, TOPK), lambda i: (i, 0)),
            pl.BlockSpec((BI, D_PAIR), lambda i: (i, 0)),
            pl.BlockSpec((BI, 1), lambda i: (i, 0)),
            wspec((1, D_PAIR)), wspec((1, D_PAIR)),
            wspec((D_PAIR, D_EDGE)), wspec((1, D_EDGE)),
            wspec((1, D_EDGE)), wspec((1, D_EDGE)),
            wspec((D_EDGE, D_EDGE)), wspec((36, D_EDGE)), wspec((1, D_EDGE)),
            wspec((1, D_EDGE)), wspec((1, D_EDGE)), wspec((1, D_EDGE)),
            wspec((L0_IN, H_MSG)), wspec((L0_IN, H_MSG)),
            wspec((D_EDGE, H_MSG)), wspec((1, H_MSG)),
        ],
        out_specs=pl.BlockSpec((BI, H_MSG), lambda i: (i, 0)),
        out_shape=jax.ShapeDtypeStruct((L, H_MSG), f32),
    )(pairk, nodek, distk, nbri, node, idx_col,
      row(g_pair_ln), row(b_pair_ln), W_e1, row(bb_e1),
      row(g_e1_ln), row(b_e1_ln),
      W_e2[:D_EDGE], W_e2[D_EDGE:D_EDGE + 36], W_e2[D_EDGE + 36:],
      row(bb_e2), row(g_e2_ln), row(b_e2_ln),
      W_msg[:L0_IN], W_msg[L0_IN:2 * L0_IN], W_msg[2 * L0_IN:], row(bb_msg))

    T, new_state, alpha = pl.pallas_call(
        _head_kernel,
        out_shape=[
            jax.ShapeDtypeStruct((L, 3), f32),
            jax.ShapeDtypeStruct((L, L0_OUT), f32),
            jax.ShapeDtypeStruct((L, 20), f32),
        ],
    )(agg, msa0, r9, tin,
      W_st, row(bb_st), W_off, row(bb_off),
      row(g_s0_ln), row(b_s0_ln), row(g_si_ln), row(b_si_ln),
      W_s0, row(bb_s0), W_si, row(bb_si),
      W_l1, row(bb_l1), W_l2, row(bb_l2),
      W_l3, row(bb_l3), W_l4, row(bb_l4),
      W_out, row(bb_out))

    return (R_in, T[None], new_state[None], alpha.reshape(1, L, 10, 2))


# speedup vs baseline: 2.8603x; 1.4653x over previous
import functools

import jax
import jax.numpy as jnp
from jax import lax
from jax.experimental import pallas as pl
from jax.experimental.pallas import tpu as pltpu
from jax.experimental.pallas import tpu_sc as plsc

B, N, L = 1, 8, 512
D_MSA, D_PAIR, D_STATE = 256, 128, 16
L0_IN, L0_OUT, D_EDGE = 32, 16, 32
D_HID = 128
H_MSG = 64
TOPK = 64

NW = 32
ROWS_PER_W = L // NW
BI = 64
NBLK = L // BI
BJC = 128
NJC = L // BJC


def _ln(x, g, b, eps=1e-5):
    m = jnp.mean(x, axis=-1, keepdims=True)
    v = jnp.var(x, axis=-1, keepdims=True)
    return (x - m) / jnp.sqrt(v + eps) * g + b


def _prep_kernel(ca_ref, caT_ref, idxc_ref, msa0_ref, state_ref,
                 g_msa_ref, b_msa_ref, g_state_ref, b_state_ref,
                 wxa_ref, wxb_ref, bbx_ref, g_node_ref, b_node_ref,
                 csum_ref, node_ref):
    ca = ca_ref[...]
    caT = caT_ref[...]
    dx = ca[:, 0:1] - caT[0:1, :]
    dy = ca[:, 1:2] - caT[1:2, :]
    dz = ca[:, 2:3] - caT[2:3, :]
    dist = jnp.sqrt(dx * dx + dy * dy + dz * dz + 1e-8)

    bits = jax.lax.bitcast_convert_type(dist, jnp.int32)
    lo0 = jnp.min(bits, axis=1, keepdims=True)
    hi0 = jnp.max(bits, axis=1, keepdims=True)

    def body(_, carry):
        lo, hi = carry
        mid = lo + (hi - lo) // 2
        cnt = jnp.sum((bits <= mid).astype(jnp.float32), axis=1, keepdims=True)
        ge = cnt >= float(TOPK)
        return jnp.where(ge, lo, mid + 1), jnp.where(ge, mid, hi)

    lo, hi = jax.lax.fori_loop(0, 31, body, (lo0, hi0))
    v = lo
    mask_lt = bits < v
    cnt_lt = jnp.sum(mask_lt.astype(jnp.float32), axis=1, keepdims=True)
    tie = bits == v
    tie_f = tie.astype(jnp.float32)
    ii = jax.lax.broadcasted_iota(jnp.int32, (L, L), 0)
    jj = jax.lax.broadcasted_iota(jnp.int32, (L, L), 1)
    M = (ii <= jj).astype(jnp.float32)
    cs_tie = jnp.dot(tie_f, M, preferred_element_type=jnp.float32)
    mask_tie = tie & (cs_tie <= (float(TOPK) - cnt_lt))
    maskf = jnp.where(mask_lt | mask_tie, 1.0, 0.0)
    csum_ref[...] = jnp.dot(maskf, M, preferred_element_type=jnp.float32)

    mln = _ln(msa0_ref[...], g_msa_ref[...], b_msa_ref[...])
    sln = _ln(state_ref[...], g_state_ref[...], b_state_ref[...])
    x = (jnp.dot(mln, wxa_ref[...], preferred_element_type=jnp.float32)
         + jnp.dot(sln, wxb_ref[...], preferred_element_type=jnp.float32)
         + bbx_ref[...])
    nodev = _ln(x, g_node_ref[...], b_node_ref[...])
    node_ref[...] = jnp.concatenate(
        [nodev, ca, idxc_ref[...],
         jnp.zeros((L, D_PAIR - L0_IN - 4), jnp.float32)], axis=1)


def _compact_kernel(c_ref, nbrf_ref, nbri_ref, flat_ref):
    jb = pl.program_id(0)
    c = c_ref[...]
    kvec1 = 1.0 + jax.lax.broadcasted_iota(
        jnp.int32, (1, 1, TOPK), 2).astype(jnp.float32)
    t = jnp.clip(kvec1 - c[:, :, None], 0.0, 1.0)

    @pl.when(jb == 0)
    def _():
        nbrf_ref[...] = jnp.zeros_like(nbrf_ref)

    nbrf_ref[...] += jnp.sum(t, axis=1)

    @pl.when(jb == NJC - 1)
    def _():
        nbri = nbrf_ref[...].astype(jnp.int32)
        nbri_ref[...] = nbri
        rowbase = jax.lax.broadcasted_iota(jnp.int32, (L, TOPK), 0) * L
        flat_ref[...] = nbri + rowbase


GPW = ROWS_PER_W // 2


def _sc_gather_kernel(flat_hbm, nbri_hbm, pairflat_hbm, node_hbm,
                      pairk_hbm, nodek_hbm,
                      idxall, nbrall, pb0, pb1, nb0, nb1,
                      semp0, semp1, semn0, semn1):
    c = lax.axis_index("c")
    s = lax.axis_index("s")
    wid = s * 2 + c
    pltpu.sync_copy(flat_hbm.at[pl.ds(wid * GPW, GPW)], idxall)
    pltpu.sync_copy(nbri_hbm.at[pl.ds(wid * GPW, GPW)], nbrall)

    def phase(idx_ref, table_hbm, out_hbm, b0, b1, s0, s1):
        bufs = (b0, b1)
        sems = (s0, s1)
        cps = [None, None]
        for p in range(GPW + 1):
            if p < GPW:
                cps[p % 2] = pltpu.async_copy(
                    table_hbm.at[idx_ref.at[p]], bufs[p % 2], sems[p % 2])
            if p >= 1:
                q = p - 1
                cps[q % 2].wait()
                base = (wid * GPW + q) * 2 * TOPK
                pltpu.sync_copy(bufs[q % 2], out_hbm.at[pl.ds(base, 2 * TOPK)])

    phase(idxall, pairflat_hbm, pairk_hbm, pb0, pb1, semp0, semp1)
    phase(nbrall, node_hbm, nodek_hbm, nb0, nb1, semn0, semn1)


def _edge_kernel(pairk_ref, nodek_ref, nodei_ref,
                 g_pair_ref, b_pair_ref, we1_ref, bbe1_ref, g_e1_ref, b_e1_ref,
                 we2a_ref, we2b_ref, we2c_ref, bbe2_ref, g_e2_ref, b_e2_ref,
                 wmi_ref, wmj_ref, wme_ref, bbm_ref,
                 agg_ref):
    x = pairk_ref[...]
    pn = _ln(x, g_pair_ref[...], b_pair_ref[...])
    e1 = (jnp.dot(pn, we1_ref[...], preferred_element_type=jnp.float32)
          + bbe1_ref[...])
    e1 = _ln(e1, g_e1_ref[...], b_e1_ref[...])

    caj = nodek_ref[:, L0_IN:L0_IN + 3]
    cai = jnp.broadcast_to(nodei_ref[:, L0_IN:L0_IN + 3][:, None, :],
                           (BI, TOPK, 3)).reshape(BI * TOPK, 3)
    dd = caj - cai
    d = jnp.sqrt(jnp.sum(dd * dd, axis=1, keepdims=True) + 1e-8)
    kidx = jax.lax.broadcasted_iota(jnp.int32, (1, 36), 1).astype(jnp.float32)
    centers = 2.0 + kidx * (20.0 / 35.0)
    sigma = 20.0 / 36.0
    rbf = jnp.exp(-(((d - centers) / sigma) ** 2))

    idxj = nodek_ref[:, L0_IN + 3:L0_IN + 4]
    idxi = jnp.broadcast_to(nodei_ref[:, L0_IN + 3:L0_IN + 4][:, None, :],
                            (BI, TOPK, 1)).reshape(BI * TOPK, 1)
    off = idxj - idxi
    seqsep = jnp.sign(off) * jnp.log(jnp.abs(off) + 1.0)

    e = (jnp.dot(e1, we2a_ref[...], preferred_element_type=jnp.float32)
         + jnp.dot(rbf, we2b_ref[...], preferred_element_type=jnp.float32)
         + seqsep * we2c_ref[...]
         + bbe2_ref[...])
    e = _ln(e, g_e2_ref[...], b_e2_ref[...])

    mi = jnp.dot(nodei_ref[:, :L0_IN], wmi_ref[...],
                 preferred_element_type=jnp.float32)
    mj = jnp.dot(nodek_ref[:, :L0_IN], wmj_ref[...],
                 preferred_element_type=jnp.float32)
    me = jnp.dot(e, wme_ref[...], preferred_element_type=jnp.float32)
    msg = jax.nn.relu(me.reshape(BI, TOPK, H_MSG)
                      + mi[:, None, :]
                      + mj.reshape(BI, TOPK, H_MSG)
                      + bbm_ref[...])
    agg_ref[...] = jnp.sum(msg, axis=1)


def _head_kernel(agg_ref, msa0_ref, r9_ref, tin_ref,
                 wst_ref, bbst_ref, woff_ref, bboff_ref,
                 g_s0_ref, b_s0_ref, g_si_ref, b_si_ref,
                 ws0_ref, bbs0_ref, wsi_ref, bbsi_ref,
                 wl1_ref, bl1_ref, wl2_ref, bl2_ref,
                 wl3_ref, bl3_ref, wl4_ref, bl4_ref,
                 wout_ref, bout_ref,
                 t_ref, ns_ref, alpha_ref):
    agg = agg_ref[...] * (1.0 / TOPK)
    ns = jnp.dot(agg, wst_ref[...],
                 preferred_element_type=jnp.float32) + bbst_ref[...]
    ns_ref[...] = ns
    off6 = jnp.dot(agg, woff_ref[...],
                   preferred_element_type=jnp.float32) + bboff_ref[...]
    delT = off6[:, 0:3] / 10.0
    r9 = r9_ref[...]
    t0 = jnp.sum(r9[:, 0:3] * delT, axis=1, keepdims=True)
    t1 = jnp.sum(r9[:, 3:6] * delT, axis=1, keepdims=True)
    t2 = jnp.sum(r9[:, 6:9] * delT, axis=1, keepdims=True)
    t_ref[...] = jnp.concatenate([t0, t1, t2], axis=1) + tin_ref[...]

    s0 = _ln(msa0_ref[...], g_s0_ref[...], b_s0_ref[...])
    si_in = _ln(ns, g_si_ref[...], b_si_ref[...])
    si = (jnp.dot(s0, ws0_ref[...], preferred_element_type=jnp.float32)
          + bbs0_ref[...]
          + jnp.dot(si_in, wsi_ref[...], preferred_element_type=jnp.float32)
          + bbsi_ref[...])
    h = jax.nn.relu(jnp.dot(jax.nn.relu(si), wl1_ref[...],
                            preferred_element_type=jnp.float32) + bl1_ref[...])
    si = si + jnp.dot(h, wl2_ref[...],
                      preferred_element_type=jnp.float32) + bl2_ref[...]
    h = jax.nn.relu(jnp.dot(jax.nn.relu(si), wl3_ref[...],
                            preferred_element_type=jnp.float32) + bl3_ref[...])
    si = si + jnp.dot(h, wl4_ref[...],
                      preferred_element_type=jnp.float32) + bl4_ref[...]
    alpha_ref[...] = (jnp.dot(jax.nn.relu(si), wout_ref[...],
                              preferred_element_type=jnp.float32)
                      + bout_ref[...])


def kernel(msa, pair, R_in, T_in, xyz, state, idx, motif_mask, top_k, g_msa_ln, b_msa_ln, g_pair_ln, b_pair_ln, g_state_ln, b_state_ln, g_node_ln, b_node_ln, g_e1_ln, b_e1_ln, g_e2_ln, b_e2_ln, g_s0_ln, b_s0_ln, g_si_ln, b_si_ln, W_x, bb_x, W_e1, bb_e1, W_e2, bb_e2, W_msg, bb_msg, W_st, bb_st, W_off, bb_off, W_s0, bb_s0, W_si, bb_si, W_l1, bb_l1, W_l2, bb_l2, W_l3, bb_l3, W_l4, bb_l4, W_out, bb_out):
    f32 = jnp.float32
    msa0 = msa[0, 0]
    ca = xyz[0, :, 1, :]
    caT = jnp.transpose(ca)
    state0 = state[0]
    idx_col = idx[0].astype(f32)[:, None]
    r9 = R_in[0].reshape(L, 9)
    tin = T_in[0]
    pairflat = pair.reshape(L * L, D_PAIR)

    row = lambda a: a.reshape(1, -1)

    csum, node = pl.pallas_call(
        _prep_kernel,
        out_shape=[
            jax.ShapeDtypeStruct((L, L), f32),
            jax.ShapeDtypeStruct((L, D_PAIR), f32),
        ],
    )(ca, caT, idx_col, msa0, state0,
      row(g_msa_ln), row(b_msa_ln), row(g_state_ln), row(b_state_ln),
      W_x[:D_MSA], W_x[D_MSA:], row(bb_x), row(g_node_ln), row(b_node_ln))

    nbrf, nbri, flatidx = pl.pallas_call(
        _compact_kernel,
        grid=(NJC,),
        in_specs=[
            pl.BlockSpec((L, BJC), lambda j: (0, j)),
        ],
        out_specs=[
            pl.BlockSpec((L, TOPK), lambda j: (0, 0)),
            pl.BlockSpec((L, TOPK), lambda j: (0, 0)),
            pl.BlockSpec((L, TOPK), lambda j: (0, 0)),
        ],
        out_shape=[
            jax.ShapeDtypeStruct((L, TOPK), f32),
            jax.ShapeDtypeStruct((L, TOPK), jnp.int32),
            jax.ShapeDtypeStruct((L, TOPK), jnp.int32),
        ],
    )(csum)
    flat2 = flatidx.reshape(L // 2, 2 * TOPK)
    nbr2 = nbri.reshape(L // 2, 2 * TOPK)

    sc_gather = functools.partial(
        pl.kernel,
        mesh=plsc.VectorSubcoreMesh(core_axis_name="c", subcore_axis_name="s"),
        out_type=[
            jax.ShapeDtypeStruct((L * TOPK, D_PAIR), f32),
            jax.ShapeDtypeStruct((L * TOPK, D_PAIR), f32),
        ],
        scratch_types=[
            pltpu.VMEM((GPW, 2 * TOPK), jnp.int32),
            pltpu.VMEM((GPW, 2 * TOPK), jnp.int32),
            pltpu.VMEM((2 * TOPK, D_PAIR), f32),
            pltpu.VMEM((2 * TOPK, D_PAIR), f32),
            pltpu.VMEM((2 * TOPK, D_PAIR), f32),
            pltpu.VMEM((2 * TOPK, D_PAIR), f32),
            pltpu.SemaphoreType.DMA,
            pltpu.SemaphoreType.DMA,
            pltpu.SemaphoreType.DMA,
            pltpu.SemaphoreType.DMA,
        ],
    )(_sc_gather_kernel)
    pairk, nodek = sc_gather(flat2, nbr2, pairflat, node)

    wspec = lambda shp: pl.BlockSpec(shp, lambda i: (0,) * len(shp))
    agg = pl.pallas_call(
        _edge_kernel,
        grid=(NBLK,),
        in_specs=[
            pl.BlockSpec((BI * TOPK, D_PAIR), lambda i: (i, 0)),
            pl.BlockSpec((BI * TOPK, D_PAIR), lambda i: (i, 0)),
            pl.BlockSpec((BI, D_PAIR), lambda i: (i, 0)),
            wspec((1, D_PAIR)), wspec((1, D_PAIR)),
            wspec((D_PAIR, D_EDGE)), wspec((1, D_EDGE)),
            wspec((1, D_EDGE)), wspec((1, D_EDGE)),
            wspec((D_EDGE, D_EDGE)), wspec((36, D_EDGE)), wspec((1, D_EDGE)),
            wspec((1, D_EDGE)), wspec((1, D_EDGE)), wspec((1, D_EDGE)),
            wspec((L0_IN, H_MSG)), wspec((L0_IN, H_MSG)),
            wspec((D_EDGE, H_MSG)), wspec((1, H_MSG)),
        ],
        out_specs=pl.BlockSpec((BI, H_MSG), lambda i: (i, 0)),
        out_shape=jax.ShapeDtypeStruct((L, H_MSG), f32),
    )(pairk, nodek, node,
      row(g_pair_ln), row(b_pair_ln), W_e1, row(bb_e1),
      row(g_e1_ln), row(b_e1_ln),
      W_e2[:D_EDGE], W_e2[D_EDGE:D_EDGE + 36], W_e2[D_EDGE + 36:],
      row(bb_e2), row(g_e2_ln), row(b_e2_ln),
      W_msg[:L0_IN], W_msg[L0_IN:2 * L0_IN], W_msg[2 * L0_IN:], row(bb_msg))

    T, new_state, alpha = pl.pallas_call(
        _head_kernel,
        out_shape=[
            jax.ShapeDtypeStruct((L, 3), f32),
            jax.ShapeDtypeStruct((L, L0_OUT), f32),
            jax.ShapeDtypeStruct((L, 20), f32),
        ],
    )(agg, msa0, r9, tin,
      W_st, row(bb_st), W_off, row(bb_off),
      row(g_s0_ln), row(b_s0_ln), row(g_si_ln), row(b_si_ln),
      W_s0, row(bb_s0), W_si, row(bb_si),
      W_l1, row(bb_l1), W_l2, row(bb_l2),
      W_l3, row(bb_l3), W_l4, row(bb_l4),
      W_out, row(bb_out))

    return (R_in, T[None], new_state[None], alpha.reshape(1, L, 10, 2))


# speedup vs baseline: 2.9258x; 1.0229x over previous
import functools

import jax
import jax.numpy as jnp
from jax import lax
from jax.experimental import pallas as pl
from jax.experimental.pallas import tpu as pltpu
from jax.experimental.pallas import tpu_sc as plsc

B, N, L = 1, 8, 512
D_MSA, D_PAIR, D_STATE = 256, 128, 16
L0_IN, L0_OUT, D_EDGE = 32, 16, 32
D_HID = 128
H_MSG = 64
TOPK = 64

NW = 32
ROWS_PER_W = L // NW
BI = 64
NBLK = L // BI
BJC = 128
NJC = L // BJC


def _ln(x, g, b, eps=1e-5):
    m = jnp.mean(x, axis=-1, keepdims=True)
    v = jnp.var(x, axis=-1, keepdims=True)
    return (x - m) / jnp.sqrt(v + eps) * g + b


def _prep_kernel(ca_ref, caT_ref, idxc_ref, msa0_ref, state_ref,
                 g_msa_ref, b_msa_ref, g_state_ref, b_state_ref,
                 wxa_ref, wxb_ref, bbx_ref, g_node_ref, b_node_ref,
                 csum_ref, node_ref):
    ca = ca_ref[...]
    caT = caT_ref[...]
    dx = ca[:, 0:1] - caT[0:1, :]
    dy = ca[:, 1:2] - caT[1:2, :]
    dz = ca[:, 2:3] - caT[2:3, :]
    dist = jnp.sqrt(dx * dx + dy * dy + dz * dz + 1e-8)

    bits = jax.lax.bitcast_convert_type(dist, jnp.int32)
    lo0 = jnp.min(bits, axis=1, keepdims=True)
    hi0 = jnp.max(bits, axis=1, keepdims=True)

    def body(_, carry):
        lo, hi = carry
        mid = lo + (hi - lo) // 2
        cnt = jnp.sum((bits <= mid).astype(jnp.float32), axis=1, keepdims=True)
        ge = cnt >= float(TOPK)
        return jnp.where(ge, lo, mid + 1), jnp.where(ge, mid, hi)

    lo, hi = jax.lax.fori_loop(0, 31, body, (lo0, hi0))
    v = lo
    mask_lt = bits < v
    cnt_lt = jnp.sum(mask_lt.astype(jnp.float32), axis=1, keepdims=True)
    tie = bits == v
    tie_f = tie.astype(jnp.float32)
    ii = jax.lax.broadcasted_iota(jnp.int32, (L, L), 0)
    jj = jax.lax.broadcasted_iota(jnp.int32, (L, L), 1)
    M = (ii <= jj).astype(jnp.float32)
    cs_tie = jnp.dot(tie_f, M, preferred_element_type=jnp.float32)
    mask_tie = tie & (cs_tie <= (float(TOPK) - cnt_lt))
    maskf = jnp.where(mask_lt | mask_tie, 1.0, 0.0)
    csum_ref[...] = jnp.dot(maskf, M, preferred_element_type=jnp.float32)

    mln = _ln(msa0_ref[...], g_msa_ref[...], b_msa_ref[...])
    sln = _ln(state_ref[...], g_state_ref[...], b_state_ref[...])
    x = (jnp.dot(mln, wxa_ref[...], preferred_element_type=jnp.float32)
         + jnp.dot(sln, wxb_ref[...], preferred_element_type=jnp.float32)
         + bbx_ref[...])
    nodev = _ln(x, g_node_ref[...], b_node_ref[...])
    node_ref[...] = jnp.concatenate(
        [nodev, ca, idxc_ref[...],
         jnp.zeros((L, D_PAIR - L0_IN - 4), jnp.float32)], axis=1)


def _compact_kernel(c_ref, nbrf_ref, nbri_ref, flat_ref):
    jb = pl.program_id(0)
    c = c_ref[...]
    kvec1 = 1.0 + jax.lax.broadcasted_iota(
        jnp.int32, (1, 1, TOPK), 2).astype(jnp.float32)
    t = jnp.clip(kvec1 - c[:, :, None], 0.0, 1.0)

    @pl.when(jb == 0)
    def _():
        nbrf_ref[...] = jnp.zeros_like(nbrf_ref)

    nbrf_ref[...] += jnp.sum(t, axis=1)

    @pl.when(jb == NJC - 1)
    def _():
        nbri = nbrf_ref[...].astype(jnp.int32)
        nbri_ref[...] = nbri
        rowbase = jax.lax.broadcasted_iota(jnp.int32, (L, TOPK), 0) * L
        flat_ref[...] = nbri + rowbase


GPW = ROWS_PER_W // 2


def _sc_gather_kernel(flat_hbm, nbri_hbm, pairflat_hbm, node_hbm,
                      pairk_hbm, nodek_hbm,
                      idxall, nbrall, pb0, pb1, nb0, nb1,
                      semp0, semp1, semn0, semn1):
    c = lax.axis_index("c")
    s = lax.axis_index("s")
    wid = s * 2 + c
    pltpu.sync_copy(flat_hbm.at[pl.ds(wid * GPW, GPW)], idxall)
    pltpu.sync_copy(nbri_hbm.at[pl.ds(wid * GPW, GPW)], nbrall)

    pbufs, psems = (pb0, pb1), (semp0, semp1)
    nbufs, nsems = (nb0, nb1), (semn0, semn1)
    pcps = [None, None]
    ncps = [None, None]
    for p in range(GPW + 1):
        if p < GPW:
            pcps[p % 2] = pltpu.async_copy(
                pairflat_hbm.at[idxall.at[p]], pbufs[p % 2], psems[p % 2])
            ncps[p % 2] = pltpu.async_copy(
                node_hbm.at[nbrall.at[p]], nbufs[p % 2], nsems[p % 2])
        if p >= 1:
            q = p - 1
            base = (wid * GPW + q) * 2 * TOPK
            pcps[q % 2].wait()
            pltpu.sync_copy(pbufs[q % 2], pairk_hbm.at[pl.ds(base, 2 * TOPK)])
            ncps[q % 2].wait()
            pltpu.sync_copy(nbufs[q % 2], nodek_hbm.at[pl.ds(base, 2 * TOPK)])


def _edge_kernel(pairk_ref, nodek_ref, nodei_ref,
                 g_pair_ref, b_pair_ref, we1_ref, bbe1_ref, g_e1_ref, b_e1_ref,
                 we2a_ref, we2b_ref, we2c_ref, bbe2_ref, g_e2_ref, b_e2_ref,
                 wmi_ref, wmj_ref, wme_ref, bbm_ref,
                 agg_ref):
    x = pairk_ref[...]
    pn = _ln(x, g_pair_ref[...], b_pair_ref[...])
    e1 = (jnp.dot(pn, we1_ref[...], preferred_element_type=jnp.float32)
          + bbe1_ref[...])
    e1 = _ln(e1, g_e1_ref[...], b_e1_ref[...])

    caj = nodek_ref[:, L0_IN:L0_IN + 3]
    cai = jnp.broadcast_to(nodei_ref[:, L0_IN:L0_IN + 3][:, None, :],
                           (BI, TOPK, 3)).reshape(BI * TOPK, 3)
    dd = caj - cai
    d = jnp.sqrt(jnp.sum(dd * dd, axis=1, keepdims=True) + 1e-8)
    kidx = jax.lax.broadcasted_iota(jnp.int32, (1, 36), 1).astype(jnp.float32)
    centers = 2.0 + kidx * (20.0 / 35.0)
    sigma = 20.0 / 36.0
    rbf = jnp.exp(-(((d - centers) / sigma) ** 2))

    idxj = nodek_ref[:, L0_IN + 3:L0_IN + 4]
    idxi = jnp.broadcast_to(nodei_ref[:, L0_IN + 3:L0_IN + 4][:, None, :],
                            (BI, TOPK, 1)).reshape(BI * TOPK, 1)
    off = idxj - idxi
    seqsep = jnp.sign(off) * jnp.log(jnp.abs(off) + 1.0)

    e = (jnp.dot(e1, we2a_ref[...], preferred_element_type=jnp.float32)
         + jnp.dot(rbf, we2b_ref[...], preferred_element_type=jnp.float32)
         + seqsep * we2c_ref[...]
         + bbe2_ref[...])
    e = _ln(e, g_e2_ref[...], b_e2_ref[...])

    mi = jnp.dot(nodei_ref[:, :L0_IN], wmi_ref[...],
                 preferred_element_type=jnp.float32)
    mj = jnp.dot(nodek_ref[:, :L0_IN], wmj_ref[...],
                 preferred_element_type=jnp.float32)
    me = jnp.dot(e, wme_ref[...], preferred_element_type=jnp.float32)
    msg = jax.nn.relu(me.reshape(BI, TOPK, H_MSG)
                      + mi[:, None, :]
                      + mj.reshape(BI, TOPK, H_MSG)
                      + bbm_ref[...])
    agg_ref[...] = jnp.sum(msg, axis=1)


def _head_kernel(agg_ref, msa0_ref, r9_ref, tin_ref,
                 wst_ref, bbst_ref, woff_ref, bboff_ref,
                 g_s0_ref, b_s0_ref, g_si_ref, b_si_ref,
                 ws0_ref, bbs0_ref, wsi_ref, bbsi_ref,
                 wl1_ref, bl1_ref, wl2_ref, bl2_ref,
                 wl3_ref, bl3_ref, wl4_ref, bl4_ref,
                 wout_ref, bout_ref,
                 t_ref, ns_ref, alpha_ref):
    agg = agg_ref[...] * (1.0 / TOPK)
    ns = jnp.dot(agg, wst_ref[...],
                 preferred_element_type=jnp.float32) + bbst_ref[...]
    ns_ref[...] = ns
    off6 = jnp.dot(agg, woff_ref[...],
                   preferred_element_type=jnp.float32) + bboff_ref[...]
    delT = off6[:, 0:3] / 10.0
    r9 = r9_ref[...]
    t0 = jnp.sum(r9[:, 0:3] * delT, axis=1, keepdims=True)
    t1 = jnp.sum(r9[:, 3:6] * delT, axis=1, keepdims=True)
    t2 = jnp.sum(r9[:, 6:9] * delT, axis=1, keepdims=True)
    t_ref[...] = jnp.concatenate([t0, t1, t2], axis=1) + tin_ref[...]

    s0 = _ln(msa0_ref[...], g_s0_ref[...], b_s0_ref[...])
    si_in = _ln(ns, g_si_ref[...], b_si_ref[...])
    si = (jnp.dot(s0, ws0_ref[...], preferred_element_type=jnp.float32)
          + bbs0_ref[...]
          + jnp.dot(si_in, wsi_ref[...], preferred_element_type=jnp.float32)
          + bbsi_ref[...])
    h = jax.nn.relu(jnp.dot(jax.nn.relu(si), wl1_ref[...],
                            preferred_element_type=jnp.float32) + bl1_ref[...])
    si = si + jnp.dot(h, wl2_ref[...],
                      preferred_element_type=jnp.float32) + bl2_ref[...]
    h = jax.nn.relu(jnp.dot(jax.nn.relu(si), wl3_ref[...],
                            preferred_element_type=jnp.float32) + bl3_ref[...])
    si = si + jnp.dot(h, wl4_ref[...],
                      preferred_element_type=jnp.float32) + bl4_ref[...]
    alpha_ref[...] = (jnp.dot(jax.nn.relu(si), wout_ref[...],
                              preferred_element_type=jnp.float32)
                      + bout_ref[...])


def kernel(msa, pair, R_in, T_in, xyz, state, idx, motif_mask, top_k, g_msa_ln, b_msa_ln, g_pair_ln, b_pair_ln, g_state_ln, b_state_ln, g_node_ln, b_node_ln, g_e1_ln, b_e1_ln, g_e2_ln, b_e2_ln, g_s0_ln, b_s0_ln, g_si_ln, b_si_ln, W_x, bb_x, W_e1, bb_e1, W_e2, bb_e2, W_msg, bb_msg, W_st, bb_st, W_off, bb_off, W_s0, bb_s0, W_si, bb_si, W_l1, bb_l1, W_l2, bb_l2, W_l3, bb_l3, W_l4, bb_l4, W_out, bb_out):
    f32 = jnp.float32
    msa0 = msa[0, 0]
    ca = xyz[0, :, 1, :]
    caT = jnp.transpose(ca)
    state0 = state[0]
    idx_col = idx[0].astype(f32)[:, None]
    r9 = R_in[0].reshape(L, 9)
    tin = T_in[0]
    pairflat = pair.reshape(L * L, D_PAIR)

    row = lambda a: a.reshape(1, -1)

    csum, node = pl.pallas_call(
        _prep_kernel,
        out_shape=[
            jax.ShapeDtypeStruct((L, L), f32),
            jax.ShapeDtypeStruct((L, D_PAIR), f32),
        ],
    )(ca, caT, idx_col, msa0, state0,
      row(g_msa_ln), row(b_msa_ln), row(g_state_ln), row(b_state_ln),
      W_x[:D_MSA], W_x[D_MSA:], row(bb_x), row(g_node_ln), row(b_node_ln))

    nbrf, nbri, flatidx = pl.pallas_call(
        _compact_kernel,
        grid=(NJC,),
        in_specs=[
            pl.BlockSpec((L, BJC), lambda j: (0, j)),
        ],
        out_specs=[
            pl.BlockSpec((L, TOPK), lambda j: (0, 0)),
            pl.BlockSpec((L, TOPK), lambda j: (0, 0)),
            pl.BlockSpec((L, TOPK), lambda j: (0, 0)),
        ],
        out_shape=[
            jax.ShapeDtypeStruct((L, TOPK), f32),
            jax.ShapeDtypeStruct((L, TOPK), jnp.int32),
            jax.ShapeDtypeStruct((L, TOPK), jnp.int32),
        ],
    )(csum)
    flat2 = flatidx.reshape(L // 2, 2 * TOPK)
    nbr2 = nbri.reshape(L // 2, 2 * TOPK)

    sc_gather = functools.partial(
        pl.kernel,
        mesh=plsc.VectorSubcoreMesh(core_axis_name="c", subcore_axis_name="s"),
        out_type=[
            jax.ShapeDtypeStruct((L * TOPK, D_PAIR), f32),
            jax.ShapeDtypeStruct((L * TOPK, D_PAIR), f32),
        ],
        scratch_types=[
            pltpu.VMEM((GPW, 2 * TOPK), jnp.int32),
            pltpu.VMEM((GPW, 2 * TOPK), jnp.int32),
            pltpu.VMEM((2 * TOPK, D_PAIR), f32),
            pltpu.VMEM((2 * TOPK, D_PAIR), f32),
            pltpu.VMEM((2 * TOPK, D_PAIR), f32),
            pltpu.VMEM((2 * TOPK, D_PAIR), f32),
            pltpu.SemaphoreType.DMA,
            pltpu.SemaphoreType.DMA,
            pltpu.SemaphoreType.DMA,
            pltpu.SemaphoreType.DMA,
        ],
    )(_sc_gather_kernel)
    pairk, nodek = sc_gather(flat2, nbr2, pairflat, node)

    wspec = lambda shp: pl.BlockSpec(shp, lambda i: (0,) * len(shp))
    agg = pl.pallas_call(
        _edge_kernel,
        grid=(NBLK,),
        in_specs=[
            pl.BlockSpec((BI * TOPK, D_PAIR), lambda i: (i, 0)),
            pl.BlockSpec((BI * TOPK, D_PAIR), lambda i: (i, 0)),
            pl.BlockSpec((BI, D_PAIR), lambda i: (i, 0)),
            wspec((1, D_PAIR)), wspec((1, D_PAIR)),
            wspec((D_PAIR, D_EDGE)), wspec((1, D_EDGE)),
            wspec((1, D_EDGE)), wspec((1, D_EDGE)),
            wspec((D_EDGE, D_EDGE)), wspec((36, D_EDGE)), wspec((1, D_EDGE)),
            wspec((1, D_EDGE)), wspec((1, D_EDGE)), wspec((1, D_EDGE)),
            wspec((L0_IN, H_MSG)), wspec((L0_IN, H_MSG)),
            wspec((D_EDGE, H_MSG)), wspec((1, H_MSG)),
        ],
        out_specs=pl.BlockSpec((BI, H_MSG), lambda i: (i, 0)),
        out_shape=jax.ShapeDtypeStruct((L, H_MSG), f32),
    )(pairk, nodek, node,
      row(g_pair_ln), row(b_pair_ln), W_e1, row(bb_e1),
      row(g_e1_ln), row(b_e1_ln),
      W_e2[:D_EDGE], W_e2[D_EDGE:D_EDGE + 36], W_e2[D_EDGE + 36:],
      row(bb_e2), row(g_e2_ln), row(b_e2_ln),
      W_msg[:L0_IN], W_msg[L0_IN:2 * L0_IN], W_msg[2 * L0_IN:], row(bb_msg))

    T, new_state, alpha = pl.pallas_call(
        _head_kernel,
        out_shape=[
            jax.ShapeDtypeStruct((L, 3), f32),
            jax.ShapeDtypeStruct((L, L0_OUT), f32),
            jax.ShapeDtypeStruct((L, 20), f32),
        ],
    )(agg, msa0, r9, tin,
      W_st, row(bb_st), W_off, row(bb_off),
      row(g_s0_ln), row(b_s0_ln), row(g_si_ln), row(b_si_ln),
      W_s0, row(bb_s0), W_si, row(bb_si),
      W_l1, row(bb_l1), W_l2, row(bb_l2),
      W_l3, row(bb_l3), W_l4, row(bb_l4),
      W_out, row(bb_out))

    return (R_in, T[None], new_state[None], alpha.reshape(1, L, 10, 2))


# speedup vs baseline: 3.0801x; 1.0527x over previous
import functools

import jax
import jax.numpy as jnp
from jax import lax
from jax.experimental import pallas as pl
from jax.experimental.pallas import tpu as pltpu
from jax.experimental.pallas import tpu_sc as plsc

B, N, L = 1, 8, 512
D_MSA, D_PAIR, D_STATE = 256, 128, 16
L0_IN, L0_OUT, D_EDGE = 32, 16, 32
D_HID = 128
H_MSG = 64
TOPK = 64

NW = 32
ROWS_PER_W = L // NW
BI = 64
NBLK = L // BI
BJC = 128
NJC = L // BJC


def _ln(x, g, b, eps=1e-5):
    m = jnp.mean(x, axis=-1, keepdims=True)
    v = jnp.var(x, axis=-1, keepdims=True)
    return (x - m) / jnp.sqrt(v + eps) * g + b


def _prep_kernel(ca_ref, caT_ref, idxc_ref, msa0_ref, state_ref,
                 g_msa_ref, b_msa_ref, g_state_ref, b_state_ref,
                 wxa_ref, wxb_ref, bbx_ref, g_node_ref, b_node_ref,
                 csum_ref, node_ref):
    ca = ca_ref[...]
    caT = caT_ref[...]
    dx = ca[:, 0:1] - caT[0:1, :]
    dy = ca[:, 1:2] - caT[1:2, :]
    dz = ca[:, 2:3] - caT[2:3, :]
    dist = jnp.sqrt(dx * dx + dy * dy + dz * dz + 1e-8)

    bits = jax.lax.bitcast_convert_type(dist, jnp.int32)
    lo0 = jnp.min(bits, axis=1, keepdims=True)
    hi0 = jnp.max(bits, axis=1, keepdims=True)

    def body(_, carry):
        lo, hi = carry
        mid = lo + (hi - lo) // 2
        cnt = jnp.sum((bits <= mid).astype(jnp.float32), axis=1, keepdims=True)
        ge = cnt >= float(TOPK)
        return jnp.where(ge, lo, mid + 1), jnp.where(ge, mid, hi)

    lo, hi = jax.lax.fori_loop(0, 31, body, (lo0, hi0))
    v = lo
    mask_lt = bits < v
    cnt_lt = jnp.sum(mask_lt.astype(jnp.float32), axis=1, keepdims=True)
    tie = bits == v
    tie_f = tie.astype(jnp.float32)
    ii = jax.lax.broadcasted_iota(jnp.int32, (L, L), 0)
    jj = jax.lax.broadcasted_iota(jnp.int32, (L, L), 1)
    M = (ii <= jj).astype(jnp.float32)
    cs_tie = jnp.dot(tie_f, M, preferred_element_type=jnp.float32)
    mask_tie = tie & (cs_tie <= (float(TOPK) - cnt_lt))
    maskf = jnp.where(mask_lt | mask_tie, 1.0, 0.0)
    csum_ref[...] = jnp.dot(maskf, M, preferred_element_type=jnp.float32)

    mln = _ln(msa0_ref[...], g_msa_ref[...], b_msa_ref[...])
    sln = _ln(state_ref[...], g_state_ref[...], b_state_ref[...])
    x = (jnp.dot(mln, wxa_ref[...], preferred_element_type=jnp.float32)
         + jnp.dot(sln, wxb_ref[...], preferred_element_type=jnp.float32)
         + bbx_ref[...])
    nodev = _ln(x, g_node_ref[...], b_node_ref[...])
    node_ref[...] = jnp.concatenate(
        [nodev, ca, idxc_ref[...],
         jnp.zeros((L, D_PAIR - L0_IN - 4), jnp.float32)], axis=1)


def _compact_kernel(c_ref, nbrf_ref, nbri_ref, flat_ref):
    jb = pl.program_id(0)
    c = c_ref[...]
    kvec1 = 1.0 + jax.lax.broadcasted_iota(
        jnp.int32, (1, 1, TOPK), 2).astype(jnp.float32)
    t = jnp.clip(kvec1 - c[:, :, None], 0.0, 1.0)

    @pl.when(jb == 0)
    def _():
        nbrf_ref[...] = jnp.zeros_like(nbrf_ref)

    nbrf_ref[...] += jnp.sum(t, axis=1)

    @pl.when(jb == NJC - 1)
    def _():
        nbri = nbrf_ref[...].astype(jnp.int32)
        nbri_ref[...] = nbri
        rowbase = jax.lax.broadcasted_iota(jnp.int32, (L, TOPK), 0) * L
        flat_ref[...] = nbri + rowbase


GPW = ROWS_PER_W // 4


def _sc_gather_kernel(flat_hbm, nbri_hbm, pairflat_hbm, node_hbm,
                      pairk_hbm, nodek_hbm,
                      idxall, nbrall, pb0, pb1, nb0, nb1,
                      semp0, semp1, semn0, semn1):
    c = lax.axis_index("c")
    s = lax.axis_index("s")
    wid = s * 2 + c
    pltpu.sync_copy(flat_hbm.at[pl.ds(wid * GPW, GPW)], idxall)
    pltpu.sync_copy(nbri_hbm.at[pl.ds(wid * GPW, GPW)], nbrall)

    pbufs, psems = (pb0, pb1), (semp0, semp1)
    nbufs, nsems = (nb0, nb1), (semn0, semn1)
    pcps = [None, None]
    ncps = [None, None]
    for p in range(GPW + 1):
        if p < GPW:
            pcps[p % 2] = pltpu.async_copy(
                pairflat_hbm.at[idxall.at[p]], pbufs[p % 2], psems[p % 2])
            ncps[p % 2] = pltpu.async_copy(
                node_hbm.at[nbrall.at[p]], nbufs[p % 2], nsems[p % 2])
        if p >= 1:
            q = p - 1
            base = (wid * GPW + q) * 2 * TOPK
            pcps[q % 2].wait()
            pltpu.sync_copy(pbufs[q % 2], pairk_hbm.at[pl.ds(base, 2 * TOPK)])
            ncps[q % 2].wait()
            pltpu.sync_copy(nbufs[q % 2], nodek_hbm.at[pl.ds(base, 2 * TOPK)])


def _edge_kernel(pairk_ref, nodek_ref, nodei_ref,
                 g_pair_ref, b_pair_ref, we1_ref, bbe1_ref, g_e1_ref, b_e1_ref,
                 we2a_ref, we2b_ref, we2c_ref, bbe2_ref, g_e2_ref, b_e2_ref,
                 wmi_ref, wmj_ref, wme_ref, bbm_ref,
                 agg_ref):
    x = pairk_ref[...]
    pn = _ln(x, g_pair_ref[...], b_pair_ref[...])
    e1 = (jnp.dot(pn, we1_ref[...], preferred_element_type=jnp.float32)
          + bbe1_ref[...])
    e1 = _ln(e1, g_e1_ref[...], b_e1_ref[...])

    caj = nodek_ref[:, L0_IN:L0_IN + 3]
    cai = jnp.broadcast_to(nodei_ref[:, L0_IN:L0_IN + 3][:, None, :],
                           (BI, TOPK, 3)).reshape(BI * TOPK, 3)
    dd = caj - cai
    d = jnp.sqrt(jnp.sum(dd * dd, axis=1, keepdims=True) + 1e-8)
    kidx = jax.lax.broadcasted_iota(jnp.int32, (1, 36), 1).astype(jnp.float32)
    centers = 2.0 + kidx * (20.0 / 35.0)
    sigma = 20.0 / 36.0
    rbf = jnp.exp(-(((d - centers) / sigma) ** 2))

    idxj = nodek_ref[:, L0_IN + 3:L0_IN + 4]
    idxi = jnp.broadcast_to(nodei_ref[:, L0_IN + 3:L0_IN + 4][:, None, :],
                            (BI, TOPK, 1)).reshape(BI * TOPK, 1)
    off = idxj - idxi
    seqsep = jnp.sign(off) * jnp.log(jnp.abs(off) + 1.0)

    e = (jnp.dot(e1, we2a_ref[...], preferred_element_type=jnp.float32)
         + jnp.dot(rbf, we2b_ref[...], preferred_element_type=jnp.float32)
         + seqsep * we2c_ref[...]
         + bbe2_ref[...])
    e = _ln(e, g_e2_ref[...], b_e2_ref[...])

    mi = jnp.dot(nodei_ref[:, :L0_IN], wmi_ref[...],
                 preferred_element_type=jnp.float32)
    mj = jnp.dot(nodek_ref[:, :L0_IN], wmj_ref[...],
                 preferred_element_type=jnp.float32)
    me = jnp.dot(e, wme_ref[...], preferred_element_type=jnp.float32)
    msg = jax.nn.relu(me.reshape(BI, TOPK, H_MSG)
                      + mi[:, None, :]
                      + mj.reshape(BI, TOPK, H_MSG)
                      + bbm_ref[...])
    agg_ref[...] = jnp.sum(msg, axis=1)


def _head_kernel(agg_ref, msa0_ref, r9_ref, tin_ref,
                 wst_ref, bbst_ref, woff_ref, bboff_ref,
                 g_s0_ref, b_s0_ref, g_si_ref, b_si_ref,
                 ws0_ref, bbs0_ref, wsi_ref, bbsi_ref,
                 wl1_ref, bl1_ref, wl2_ref, bl2_ref,
                 wl3_ref, bl3_ref, wl4_ref, bl4_ref,
                 wout_ref, bout_ref,
                 t_ref, ns_ref, alpha_ref):
    agg = agg_ref[...] * (1.0 / TOPK)
    ns = jnp.dot(agg, wst_ref[...],
                 preferred_element_type=jnp.float32) + bbst_ref[...]
    ns_ref[...] = ns
    off6 = jnp.dot(agg, woff_ref[...],
                   preferred_element_type=jnp.float32) + bboff_ref[...]
    delT = off6[:, 0:3] / 10.0
    r9 = r9_ref[...]
    t0 = jnp.sum(r9[:, 0:3] * delT, axis=1, keepdims=True)
    t1 = jnp.sum(r9[:, 3:6] * delT, axis=1, keepdims=True)
    t2 = jnp.sum(r9[:, 6:9] * delT, axis=1, keepdims=True)
    t_ref[...] = jnp.concatenate([t0, t1, t2], axis=1) + tin_ref[...]

    s0 = _ln(msa0_ref[...], g_s0_ref[...], b_s0_ref[...])
    si_in = _ln(ns, g_si_ref[...], b_si_ref[...])
    si = (jnp.dot(s0, ws0_ref[...], preferred_element_type=jnp.float32)
          + bbs0_ref[...]
          + jnp.dot(si_in, wsi_ref[...], preferred_element_type=jnp.float32)
          + bbsi_ref[...])
    h = jax.nn.relu(jnp.dot(jax.nn.relu(si), wl1_ref[...],
                            preferred_element_type=jnp.float32) + bl1_ref[...])
    si = si + jnp.dot(h, wl2_ref[...],
                      preferred_element_type=jnp.float32) + bl2_ref[...]
    h = jax.nn.relu(jnp.dot(jax.nn.relu(si), wl3_ref[...],
                            preferred_element_type=jnp.float32) + bl3_ref[...])
    si = si + jnp.dot(h, wl4_ref[...],
                      preferred_element_type=jnp.float32) + bl4_ref[...]
    alpha_ref[...] = (jnp.dot(jax.nn.relu(si), wout_ref[...],
                              preferred_element_type=jnp.float32)
                      + bout_ref[...])


def kernel(msa, pair, R_in, T_in, xyz, state, idx, motif_mask, top_k, g_msa_ln, b_msa_ln, g_pair_ln, b_pair_ln, g_state_ln, b_state_ln, g_node_ln, b_node_ln, g_e1_ln, b_e1_ln, g_e2_ln, b_e2_ln, g_s0_ln, b_s0_ln, g_si_ln, b_si_ln, W_x, bb_x, W_e1, bb_e1, W_e2, bb_e2, W_msg, bb_msg, W_st, bb_st, W_off, bb_off, W_s0, bb_s0, W_si, bb_si, W_l1, bb_l1, W_l2, bb_l2, W_l3, bb_l3, W_l4, bb_l4, W_out, bb_out):
    f32 = jnp.float32
    msa0 = msa[0, 0]
    ca = xyz[0, :, 1, :]
    caT = jnp.transpose(ca)
    state0 = state[0]
    idx_col = idx[0].astype(f32)[:, None]
    r9 = R_in[0].reshape(L, 9)
    tin = T_in[0]
    pairflat = pair.reshape(L * L, D_PAIR)

    row = lambda a: a.reshape(1, -1)

    csum, node = pl.pallas_call(
        _prep_kernel,
        out_shape=[
            jax.ShapeDtypeStruct((L, L), f32),
            jax.ShapeDtypeStruct((L, D_PAIR), f32),
        ],
    )(ca, caT, idx_col, msa0, state0,
      row(g_msa_ln), row(b_msa_ln), row(g_state_ln), row(b_state_ln),
      W_x[:D_MSA], W_x[D_MSA:], row(bb_x), row(g_node_ln), row(b_node_ln))

    nbrf, nbri, flatidx = pl.pallas_call(
        _compact_kernel,
        grid=(NJC,),
        in_specs=[
            pl.BlockSpec((L, BJC), lambda j: (0, j)),
        ],
        out_specs=[
            pl.BlockSpec((L, TOPK), lambda j: (0, 0)),
            pl.BlockSpec((L, TOPK), lambda j: (0, 0)),
            pl.BlockSpec((L, TOPK), lambda j: (0, 0)),
        ],
        out_shape=[
            jax.ShapeDtypeStruct((L, TOPK), f32),
            jax.ShapeDtypeStruct((L, TOPK), jnp.int32),
            jax.ShapeDtypeStruct((L, TOPK), jnp.int32),
        ],
    )(csum)
    flat2 = flatidx.reshape(L // 2, 2 * TOPK)
    nbr2 = nbri.reshape(L // 2, 2 * TOPK)

    sc_gather = functools.partial(
        pl.kernel,
        mesh=plsc.VectorSubcoreMesh(core_axis_name="c", subcore_axis_name="s"),
        out_type=[
            jax.ShapeDtypeStruct((L * TOPK // 2, D_PAIR), f32),
            jax.ShapeDtypeStruct((L * TOPK // 2, D_PAIR), f32),
        ],
        scratch_types=[
            pltpu.VMEM((GPW, 2 * TOPK), jnp.int32),
            pltpu.VMEM((GPW, 2 * TOPK), jnp.int32),
            pltpu.VMEM((2 * TOPK, D_PAIR), f32),
            pltpu.VMEM((2 * TOPK, D_PAIR), f32),
            pltpu.VMEM((2 * TOPK, D_PAIR), f32),
            pltpu.VMEM((2 * TOPK, D_PAIR), f32),
            pltpu.SemaphoreType.DMA,
            pltpu.SemaphoreType.DMA,
            pltpu.SemaphoreType.DMA,
            pltpu.SemaphoreType.DMA,
        ],
    )(_sc_gather_kernel)
    H2 = L // 2
    pairk0, nodek0 = sc_gather(flat2[:H2 // 2], nbr2[:H2 // 2], pairflat, node)
    pairk1, nodek1 = sc_gather(flat2[H2 // 2:], nbr2[H2 // 2:], pairflat, node)

    wspec = lambda shp: pl.BlockSpec(shp, lambda i: (0,) * len(shp))
    edge_call = lambda: pl.pallas_call(
        _edge_kernel,
        grid=(NBLK // 2,),
        in_specs=[
            pl.BlockSpec((BI * TOPK, D_PAIR), lambda i: (i, 0)),
            pl.BlockSpec((BI * TOPK, D_PAIR), lambda i: (i, 0)),
            pl.BlockSpec((BI, D_PAIR), lambda i: (i, 0)),
            wspec((1, D_PAIR)), wspec((1, D_PAIR)),
            wspec((D_PAIR, D_EDGE)), wspec((1, D_EDGE)),
            wspec((1, D_EDGE)), wspec((1, D_EDGE)),
            wspec((D_EDGE, D_EDGE)), wspec((36, D_EDGE)), wspec((1, D_EDGE)),
            wspec((1, D_EDGE)), wspec((1, D_EDGE)), wspec((1, D_EDGE)),
            wspec((L0_IN, H_MSG)), wspec((L0_IN, H_MSG)),
            wspec((D_EDGE, H_MSG)), wspec((1, H_MSG)),
        ],
        out_specs=pl.BlockSpec((BI, H_MSG), lambda i: (i, 0)),
        out_shape=jax.ShapeDtypeStruct((L // 2, H_MSG), f32),
    )
    wargs = (row(g_pair_ln), row(b_pair_ln), W_e1, row(bb_e1),
             row(g_e1_ln), row(b_e1_ln),
             W_e2[:D_EDGE], W_e2[D_EDGE:D_EDGE + 36], W_e2[D_EDGE + 36:],
             row(bb_e2), row(g_e2_ln), row(b_e2_ln),
             W_msg[:L0_IN], W_msg[L0_IN:2 * L0_IN], W_msg[2 * L0_IN:],
             row(bb_msg))
    agg0 = edge_call()(pairk0, nodek0, node[:L // 2], *wargs)
    agg1 = edge_call()(pairk1, nodek1, node[L // 2:], *wargs)
    agg = jnp.concatenate([agg0, agg1], axis=0)

    T, new_state, alpha = pl.pallas_call(
        _head_kernel,
        out_shape=[
            jax.ShapeDtypeStruct((L, 3), f32),
            jax.ShapeDtypeStruct((L, L0_OUT), f32),
            jax.ShapeDtypeStruct((L, 20), f32),
        ],
    )(agg, msa0, r9, tin,
      W_st, row(bb_st), W_off, row(bb_off),
      row(g_s0_ln), row(b_s0_ln), row(g_si_ln), row(b_si_ln),
      W_s0, row(bb_s0), W_si, row(bb_si),
      W_l1, row(bb_l1), W_l2, row(bb_l2),
      W_l3, row(bb_l3), W_l4, row(bb_l4),
      W_out, row(bb_out))

    return (R_in, T[None], new_state[None], alpha.reshape(1, L, 10, 2))
